# baseline (speedup 1.0000x reference)
import functools

import jax
import jax.numpy as jnp
from jax import lax
from jax.experimental import pallas as pl
from jax.experimental.pallas import tpu as pltpu
from jax.experimental.pallas import tpu_sc as plsc

DIM = 16
BATCH = 16384

_INFO = plsc.get_sparse_core_info()
NC = _INFO.num_cores
NS = _INFO.num_subcores
L = _INFO.num_lanes
NW = NC * NS
BPW = BATCH // NW
CH = 128
NCH = BPW // CH
GROUPS = BPW // L


def _cosine_body(e_id_hbm, o_id_hbm, mentees_hbm, mentors_hbm, out_hbm,
                 eidx_v, oidx_v, erows_v, orows_v, out_v, sem):
    wid = lax.axis_index("s") * NC + lax.axis_index("c")
    base = wid * BPW

    pltpu.sync_copy(e_id_hbm.at[wid], eidx_v)
    pltpu.sync_copy(o_id_hbm.at[wid], oidx_v)

    copies = []
    for j in range(NCH):
        copies.append(pltpu.async_copy(
            mentees_hbm.at[eidx_v.at[j]], erows_v.at[pl.ds(j * CH, CH)], sem))
        copies.append(pltpu.async_copy(
            mentors_hbm.at[oidx_v.at[j]], orows_v.at[pl.ds(j * CH, CH)], sem))
    for c in copies:
        c.wait()

    lanes = lax.iota(jnp.int32, L)

    def group(g, carry):
        row = lanes + g * L
        acc_eo = jnp.zeros((L,), jnp.float32)
        acc_ee = jnp.zeros((L,), jnp.float32)
        acc_oo = jnp.zeros((L,), jnp.float32)
        for d in range(DIM):
            dcol = jnp.full((L,), d, jnp.int32)
            ev = plsc.load_gather(erows_v, [row, dcol])
            ov = plsc.load_gather(orows_v, [row, dcol])
            acc_eo = acc_eo + ev * ov
            acc_ee = acc_ee + ev * ev
            acc_oo = acc_oo + ov * ov
        denom = acc_ee * acc_oo
        seed = jnp.int32(0x5F3759DF) - (
            lax.bitcast_convert_type(denom, jnp.int32) >> 1)
        y = lax.bitcast_convert_type(seed, jnp.float32)
        for _ in range(3):
            y = y * (jnp.float32(1.5) - jnp.float32(0.5) * denom * y * y)
        out_v[pl.ds(g * L, L)] = acc_eo * y
        return carry

    lax.fori_loop(0, GROUPS, group, jnp.int32(0))
    pltpu.sync_copy(out_v, out_hbm.at[pl.ds(base, BPW)])


_sc_cosine = functools.partial(
    pl.kernel,
    out_type=jax.ShapeDtypeStruct((BATCH,), jnp.float32),
    mesh=plsc.VectorSubcoreMesh(core_axis_name="c", subcore_axis_name="s"),
    compiler_params=pltpu.CompilerParams(
        needs_layout_passes=False, use_tc_tiling_on_sc=False),
    scratch_types=[
        pltpu.VMEM((NCH, CH), jnp.int32),
        pltpu.VMEM((NCH, CH), jnp.int32),
        pltpu.VMEM((BPW, DIM), jnp.float32),
        pltpu.VMEM((BPW, DIM), jnp.float32),
        pltpu.VMEM((BPW,), jnp.float32),
        pltpu.SemaphoreType.DMA,
    ],
)(_cosine_body)


def kernel(e_id, o_id, mentees, mentors):
    e = e_id.astype(jnp.int32).reshape(NW, NCH, CH)
    o = o_id.astype(jnp.int32).reshape(NW, NCH, CH)
    return _sc_cosine(e, o, mentees, mentors)
